# Initial kernel scaffold; baseline (speedup 1.0000x reference)
import jax
import jax.numpy as jnp
from jax.experimental import pallas as pl


def kernel(input, weight):
    raise NotImplementedError("write your pallas kernel here")


# speedup vs baseline: 4.0844x; 4.0844x over previous
import functools

import jax
import jax.numpy as jnp
from jax import lax
from jax.experimental import pallas as pl
from jax.experimental.pallas import tpu as pltpu
from jax.experimental.pallas import tpu_sc as plsc

BATCH = 4096
HIST = 50
DIM = 64
TOTAL = BATCH * HIST
NUM_CORES = 2
NUM_SUBCORES = 16
NW = NUM_CORES * NUM_SUBCORES
PER_W = TOTAL // NW
CHUNK = 128
NCH = PER_W // CHUNK


def _emb_body(idx_hbm, table_hbm, out_hbm, idx_v, rows_v, sem_g, sem_o):
    wid = lax.axis_index("s") * NUM_CORES + lax.axis_index("c")
    base = wid * PER_W

    pltpu.sync_copy(idx_hbm.at[wid], idx_v)

    def chunk(j, _):
        pltpu.async_copy(table_hbm.at[idx_v.at[j]], rows_v, sem_g).wait()
        pltpu.async_copy(rows_v, out_hbm.at[pl.ds(base + j * CHUNK, CHUNK)],
                         sem_o).wait()
        return ()

    lax.fori_loop(0, NCH, chunk, ())


@functools.partial(jax.jit, static_argnames=())
def kernel(input, weight):
    idx2d = input.reshape(NW, NCH, CHUNK)
    mesh = plsc.VectorSubcoreMesh(core_axis_name="c", subcore_axis_name="s")
    out = pl.kernel(
        _emb_body,
        out_type=jax.ShapeDtypeStruct((TOTAL, DIM), jnp.float32),
        mesh=mesh,
        scratch_types=[
            pltpu.VMEM((NCH, CHUNK), jnp.int32),
            pltpu.VMEM((CHUNK, DIM), jnp.float32),
            pltpu.SemaphoreType.DMA,
            pltpu.SemaphoreType.DMA,
        ],
        compiler_params=pltpu.CompilerParams(use_tc_tiling_on_sc=False),
    )(idx2d, weight)
    return out.reshape(BATCH, HIST, DIM)


# speedup vs baseline: 4.6596x; 1.1408x over previous
import jax
import jax.numpy as jnp
from jax import lax
from jax.experimental import pallas as pl
from jax.experimental.pallas import tpu as pltpu
from jax.experimental.pallas import tpu_sc as plsc

BATCH = 4096
HIST = 50
DIM = 64
TOTAL = BATCH * HIST
NUM_CORES = 2
NUM_SUBCORES = 16
NW = NUM_CORES * NUM_SUBCORES
PER_W = TOTAL // NW
CHUNK = 128
NCH = PER_W // CHUNK
G = 5
NGRP = NCH // G
GROUP_ROWS = G * CHUNK


def _emb_body(idx_hbm, table_hbm, out_hbm, idx_v, rows_a, rows_b,
              gs_a, gs_b, os_a, os_b):
    wid = lax.axis_index("s") * NUM_CORES + lax.axis_index("c")
    base = wid * PER_W

    pltpu.sync_copy(idx_hbm.at[wid], idx_v)

    bufs = (rows_a, rows_b)
    gsems = (gs_a, gs_b)
    osems = (os_a, os_b)

    def fire_group(g, buf, sem):
        handles = []
        for c in range(G):
            j = g * G + c
            handles.append(pltpu.async_copy(
                table_hbm.at[idx_v.at[j]],
                buf.at[pl.ds(c * CHUNK, CHUNK)], sem))
        return handles

    gh = [fire_group(0, bufs[0], gsems[0]), None]
    oh = [None, None]
    for g in range(NGRP):
        cur = g % 2
        nxt = 1 - cur
        if g + 1 < NGRP:
            if oh[nxt] is not None:
                oh[nxt].wait()
            gh[nxt] = fire_group(g + 1, bufs[nxt], gsems[nxt])
        for h in gh[cur]:
            h.wait()
        oh[cur] = pltpu.async_copy(
            bufs[cur],
            out_hbm.at[pl.ds(base + g * GROUP_ROWS, GROUP_ROWS)],
            osems[cur])
    oh[0].wait()
    oh[1].wait()


def kernel(input, weight):
    idx3d = input.reshape(NW, NCH, CHUNK)
    mesh = plsc.VectorSubcoreMesh(core_axis_name="c", subcore_axis_name="s")
    out = pl.kernel(
        _emb_body,
        out_type=jax.ShapeDtypeStruct((TOTAL, DIM), jnp.float32),
        mesh=mesh,
        scratch_types=[
            pltpu.VMEM((NCH, CHUNK), jnp.int32),
            pltpu.VMEM((GROUP_ROWS, DIM), jnp.float32),
            pltpu.VMEM((GROUP_ROWS, DIM), jnp.float32),
            pltpu.SemaphoreType.DMA,
            pltpu.SemaphoreType.DMA,
            pltpu.SemaphoreType.DMA,
            pltpu.SemaphoreType.DMA,
        ],
        compiler_params=pltpu.CompilerParams(use_tc_tiling_on_sc=False),
    )(idx3d, weight)
    return out.reshape(BATCH, HIST, DIM)
